# baseline (speedup 1.0000x reference)
import functools

import jax
import jax.numpy as jnp
from jax import lax
from jax.experimental import pallas as pl
from jax.experimental.pallas import tpu as pltpu
from jax.experimental.pallas import tpu_sc as plsc

N = 10000
E = 320000
D = 128
G = 64
NC = 2
NS = 16
NW = NC * NS
EPT = E // NW
CHUNK = 80
NCHUNK = EPT // CHUNK
RPT = N // NS
BN_INV = 1.0 / (1.0 + 1e-5) ** 0.5


def _sc_agg_body(h_hbm, src_hbm, dst_hbm, zeros_hbm, out_hbm,
                 agg_sh, src_v, dst_v, rows0, rows1, sem0, sem1, ss0, ss1):
    c = lax.axis_index("c")
    s = lax.axis_index("s")
    pltpu.sync_copy(zeros_hbm, agg_sh.at[pl.ds(s * RPT, RPT)])
    pltpu.sync_copy(src_hbm.at[c, s], src_v)
    pltpu.sync_copy(dst_hbm.at[c, s], dst_v)
    plsc.subcore_barrier()

    def gather(i, buf, sem):
        pltpu.async_copy(h_hbm.at[src_v.at[pl.ds(i * CHUNK, CHUNK)]], buf, sem)

    def gwait(i, buf, sem):
        pltpu.make_async_copy(
            h_hbm.at[src_v.at[pl.ds(i * CHUNK, CHUNK)]], buf, sem).wait()

    def scat(i, buf, sem):
        pltpu.async_copy(buf, agg_sh.at[dst_v.at[i]], sem, add=True)

    def swait(i, buf, sem):
        pltpu.make_async_copy(buf, agg_sh.at[dst_v.at[i]], sem).wait()

    gather(0, rows0, sem0)
    gwait(0, rows0, sem0)
    scat(0, rows0, ss0)
    gather(1, rows1, sem1)

    def pair(j, carry):
        k = 2 * j + 1
        gwait(k, rows1, sem1)
        scat(k, rows1, ss1)
        swait(k - 1, rows0, ss0)
        gather(k + 1, rows0, sem0)
        gwait(k + 1, rows0, sem0)
        scat(k + 1, rows0, ss0)
        swait(k, rows1, ss1)
        gather(k + 2, rows1, sem1)
        return carry

    lax.fori_loop(0, (NCHUNK - 3) // 2, pair, 0)
    k = NCHUNK - 2
    gwait(k, rows1, sem1)
    scat(k, rows1, ss1)
    swait(k - 1, rows0, ss0)
    gather(k + 1, rows0, sem0)
    gwait(k + 1, rows0, sem0)
    scat(k + 1, rows0, ss0)
    swait(k, rows1, ss1)
    swait(k + 1, rows0, ss0)
    plsc.subcore_barrier()
    pltpu.sync_copy(agg_sh.at[pl.ds(s * RPT, RPT)], out_hbm.at[c, s])


_sc_agg = pl.kernel(
    _sc_agg_body,
    out_type=jax.ShapeDtypeStruct((NC, NS, RPT, D), jnp.float32),
    mesh=plsc.VectorSubcoreMesh(core_axis_name="c", subcore_axis_name="s"),
    scratch_types=[
        pltpu.VMEM_SHARED((N, D), jnp.float32),
        pltpu.VMEM((EPT,), jnp.int32),
        pltpu.VMEM((NCHUNK, CHUNK), jnp.int32),
        pltpu.VMEM((CHUNK, D), jnp.float32),
        pltpu.VMEM((CHUNK, D), jnp.float32),
        pltpu.SemaphoreType.DMA,
        pltpu.SemaphoreType.DMA,
        pltpu.SemaphoreType.DMA,
        pltpu.SemaphoreType.DMA,
    ],
)


def _tc_layer_body(h_ref, p0_ref, p1_ref, w1_ref, b1_ref, g_ref, be_ref,
                   w2_ref, b2_ref, o_ref):
    z = h_ref[...] + p0_ref[...] + p1_ref[...]
    z = jnp.dot(z, w1_ref[...], preferred_element_type=jnp.float32)
    z = (z + b1_ref[...]) * (g_ref[...] * BN_INV) + be_ref[...]
    z = jnp.maximum(z, 0.0)
    z = jnp.dot(z, w2_ref[...], preferred_element_type=jnp.float32)
    o_ref[...] = jnp.maximum(z + b2_ref[...], 0.0)


def _tc_layer(h, p0, p1, w1, b1, g, be, w2, b2):
    nb = 10
    blk = N // nb
    row_spec = pl.BlockSpec((blk, D), lambda i: (i, 0))
    full = pl.BlockSpec((D, D), lambda i: (0, 0))
    vec = pl.BlockSpec((1, D), lambda i: (0, 0))
    return pl.pallas_call(
        _tc_layer_body,
        grid=(nb,),
        in_specs=[row_spec, row_spec, row_spec, full, vec, vec, vec, full, vec],
        out_specs=row_spec,
        out_shape=jax.ShapeDtypeStruct((N, D), jnp.float32),
    )(h, p0, p1, w1, b1.reshape(1, D), g.reshape(1, D), be.reshape(1, D),
      w2, b2.reshape(1, D))


_NB3 = 10
_BLK3 = N // _NB3


def _tc_layer3_pool_head_body(h_ref, p0_ref, p1_ref, w1_ref, b1_ref, g_ref,
                              be_ref, w2_ref, b2_ref, batch_ref, hw1_ref,
                              hb1_ref, hw2_ref, hb2_ref, o_ref,
                              sums_ref, cnt_ref):
    i = pl.program_id(0)

    @pl.when(i == 0)
    def _():
        sums_ref[...] = jnp.zeros_like(sums_ref)
        cnt_ref[...] = jnp.zeros_like(cnt_ref)

    z = h_ref[...] + p0_ref[...] + p1_ref[...]
    z = jnp.dot(z, w1_ref[...], preferred_element_type=jnp.float32)
    z = (z + b1_ref[...]) * (g_ref[...] * BN_INV) + be_ref[...]
    z = jnp.maximum(z, 0.0)
    z = jnp.dot(z, w2_ref[...], preferred_element_type=jnp.float32)
    z = jnp.maximum(z + b2_ref[...], 0.0)

    gids = lax.broadcasted_iota(jnp.int32, (G, _BLK3), 0)
    onehot = (batch_ref[0] == gids).astype(jnp.float32)
    sums_ref[...] += jnp.dot(onehot, z, preferred_element_type=jnp.float32)
    cnt_ref[...] += jnp.sum(onehot, axis=1, keepdims=True)

    @pl.when(i == _NB3 - 1)
    def _():
        pooled = sums_ref[...] / jnp.maximum(cnt_ref[...], 1.0)
        zz = jnp.dot(pooled, hw1_ref[...], preferred_element_type=jnp.float32)
        zz = jnp.maximum(zz + hb1_ref[...], 0.0)
        zz = jnp.dot(zz, hw2_ref[...], preferred_element_type=jnp.float32)
        o_ref[...] = zz + hb2_ref[...]


def _tc_layer3_pool_head(h, p0, p1, w1, b1, g, be, w2, b2, batch,
                         hw1, hb1, hw2, hb2):
    row_spec = pl.BlockSpec((_BLK3, D), lambda i: (i, 0))
    full = pl.BlockSpec((D, D), lambda i: (0, 0))
    vec = pl.BlockSpec((1, D), lambda i: (0, 0))
    return pl.pallas_call(
        _tc_layer3_pool_head_body,
        grid=(_NB3,),
        in_specs=[row_spec, row_spec, row_spec, full, vec, vec, vec, full,
                  vec,
                  pl.BlockSpec((1, 1, _BLK3), lambda i: (i, 0, 0)),
                  full, vec,
                  pl.BlockSpec((D, 10), lambda i: (0, 0)),
                  pl.BlockSpec((1, 10), lambda i: (0, 0))],
        out_specs=pl.BlockSpec((G, 10), lambda i: (0, 0)),
        out_shape=jax.ShapeDtypeStruct((G, 10), jnp.float32),
        scratch_shapes=[pltpu.VMEM((G, D), jnp.float32),
                        pltpu.VMEM((G, 1), jnp.float32)],
    )(h, p0, p1, w1, b1.reshape(1, D), g.reshape(1, D), be.reshape(1, D),
      w2, b2.reshape(1, D), batch.reshape(_NB3, 1, _BLK3),
      hw1, hb1.reshape(1, D), hw2, hb2.reshape(1, 10))


@jax.jit
def kernel(x, edge_index, batch, conv_W1, conv_b1, conv_gamma, conv_beta,
           conv_W2, conv_b2, head_W1, head_b1, head_W2, head_b2):
    src = edge_index[0].reshape(NC, NS, EPT)
    dst = edge_index[1].reshape(NC, NS, NCHUNK, CHUNK)
    zeros = jnp.zeros((RPT, D), dtype=jnp.float32)
    h0 = x
    for i in range(2):
        p = _sc_agg(h0, src, dst, zeros).reshape(NC, N, D)
        h0 = _tc_layer(h0, p[0], p[1], conv_W1[i], conv_b1[i],
                       conv_gamma[i], conv_beta[i], conv_W2[i], conv_b2[i])
    p = _sc_agg(h0, src, dst, zeros).reshape(NC, N, D)
    return _tc_layer3_pool_head(h0, p[0], p[1], conv_W1[2], conv_b1[2],
                                conv_gamma[2], conv_beta[2], conv_W2[2],
                                conv_b2[2], batch, head_W1, head_b1,
                                head_W2, head_b2)


# speedup vs baseline: 1.2320x; 1.2320x over previous
import functools

import jax
import jax.numpy as jnp
from jax import lax
from jax.experimental import pallas as pl
from jax.experimental.pallas import tpu as pltpu
from jax.experimental.pallas import tpu_sc as plsc

N = 10000
E = 320000
D = 128
G = 64
NC = 2
NS = 16
NW = NC * NS
EPT = E // NW
CHUNK = 80
NCHUNK = EPT // CHUNK
RPT = N // NS
BN_INV = 1.0 / (1.0 + 1e-5) ** 0.5


def _sc_agg_body(h_hbm, src_hbm, dst_hbm, zeros_hbm, out_hbm,
                 agg_sh, src_v, dst_v, rows0, rows1, sem0, sem1):
    c = lax.axis_index("c")
    s = lax.axis_index("s")
    pltpu.sync_copy(zeros_hbm, agg_sh.at[pl.ds(s * RPT, RPT)])
    pltpu.sync_copy(src_hbm.at[c, s], src_v)
    pltpu.sync_copy(dst_hbm.at[c, s], dst_v)
    plsc.subcore_barrier()

    def gather(i, buf, sem):
        pltpu.async_copy(h_hbm.at[src_v.at[pl.ds(i * CHUNK, CHUNK)]], buf, sem)

    def gwait(i, buf, sem):
        pltpu.make_async_copy(
            h_hbm.at[src_v.at[pl.ds(i * CHUNK, CHUNK)]], buf, sem).wait()

    def scat(i, buf):
        pltpu.sync_copy(buf, agg_sh.at[dst_v.at[i]], add=True)

    gather(0, rows0, sem0)

    def pair(j, carry):
        i0 = 2 * j
        gather(i0 + 1, rows1, sem1)
        gwait(i0, rows0, sem0)
        scat(i0, rows0)
        gather(i0 + 2, rows0, sem0)
        gwait(i0 + 1, rows1, sem1)
        scat(i0 + 1, rows1)
        return carry

    lax.fori_loop(0, (NCHUNK - 1) // 2, pair, 0)
    gwait(NCHUNK - 1, rows0, sem0)
    scat(NCHUNK - 1, rows0)
    plsc.subcore_barrier()
    pltpu.sync_copy(agg_sh.at[pl.ds(s * RPT, RPT)], out_hbm.at[c, s])


_sc_agg = pl.kernel(
    _sc_agg_body,
    out_type=jax.ShapeDtypeStruct((NC, NS, RPT, D), jnp.float32),
    mesh=plsc.VectorSubcoreMesh(core_axis_name="c", subcore_axis_name="s"),
    scratch_types=[
        pltpu.VMEM_SHARED((N, D), jnp.float32),
        pltpu.VMEM((EPT,), jnp.int32),
        pltpu.VMEM((NCHUNK, CHUNK), jnp.int32),
        pltpu.VMEM((CHUNK, D), jnp.float32),
        pltpu.VMEM((CHUNK, D), jnp.float32),
        pltpu.SemaphoreType.DMA,
        pltpu.SemaphoreType.DMA,
    ],
)


def _tc_layer_body(h_ref, p0_ref, p1_ref, w1_ref, b1_ref, g_ref, be_ref,
                   w2_ref, b2_ref, o_ref):
    z = h_ref[...] + p0_ref[...] + p1_ref[...]
    z = jnp.dot(z, w1_ref[...], preferred_element_type=jnp.float32)
    z = (z + b1_ref[...]) * (g_ref[...] * BN_INV) + be_ref[...]
    z = jnp.maximum(z, 0.0)
    z = jnp.dot(z, w2_ref[...], preferred_element_type=jnp.float32)
    o_ref[...] = jnp.maximum(z + b2_ref[...], 0.0)


def _tc_layer(h, p0, p1, w1, b1, g, be, w2, b2):
    nb = 10
    blk = N // nb
    row_spec = pl.BlockSpec((blk, D), lambda i: (i, 0))
    full = pl.BlockSpec((D, D), lambda i: (0, 0))
    vec = pl.BlockSpec((1, D), lambda i: (0, 0))
    return pl.pallas_call(
        _tc_layer_body,
        grid=(nb,),
        in_specs=[row_spec, row_spec, row_spec, full, vec, vec, vec, full, vec],
        out_specs=row_spec,
        out_shape=jax.ShapeDtypeStruct((N, D), jnp.float32),
    )(h, p0, p1, w1, b1.reshape(1, D), g.reshape(1, D), be.reshape(1, D),
      w2, b2.reshape(1, D))


_NB3 = 10
_BLK3 = N // _NB3


def _tc_layer3_pool_head_body(h_ref, p0_ref, p1_ref, w1_ref, b1_ref, g_ref,
                              be_ref, w2_ref, b2_ref, batch_ref, hw1_ref,
                              hb1_ref, hw2_ref, hb2_ref, o_ref,
                              sums_ref, cnt_ref):
    i = pl.program_id(0)

    @pl.when(i == 0)
    def _():
        sums_ref[...] = jnp.zeros_like(sums_ref)
        cnt_ref[...] = jnp.zeros_like(cnt_ref)

    z = h_ref[...] + p0_ref[...] + p1_ref[...]
    z = jnp.dot(z, w1_ref[...], preferred_element_type=jnp.float32)
    z = (z + b1_ref[...]) * (g_ref[...] * BN_INV) + be_ref[...]
    z = jnp.maximum(z, 0.0)
    z = jnp.dot(z, w2_ref[...], preferred_element_type=jnp.float32)
    z = jnp.maximum(z + b2_ref[...], 0.0)

    gids = lax.broadcasted_iota(jnp.int32, (G, _BLK3), 0)
    onehot = (batch_ref[0] == gids).astype(jnp.float32)
    sums_ref[...] += jnp.dot(onehot, z, preferred_element_type=jnp.float32)
    cnt_ref[...] += jnp.sum(onehot, axis=1, keepdims=True)

    @pl.when(i == _NB3 - 1)
    def _():
        pooled = sums_ref[...] / jnp.maximum(cnt_ref[...], 1.0)
        zz = jnp.dot(pooled, hw1_ref[...], preferred_element_type=jnp.float32)
        zz = jnp.maximum(zz + hb1_ref[...], 0.0)
        zz = jnp.dot(zz, hw2_ref[...], preferred_element_type=jnp.float32)
        o_ref[...] = zz + hb2_ref[...]


def _tc_layer3_pool_head(h, p0, p1, w1, b1, g, be, w2, b2, batch,
                         hw1, hb1, hw2, hb2):
    row_spec = pl.BlockSpec((_BLK3, D), lambda i: (i, 0))
    full = pl.BlockSpec((D, D), lambda i: (0, 0))
    vec = pl.BlockSpec((1, D), lambda i: (0, 0))
    return pl.pallas_call(
        _tc_layer3_pool_head_body,
        grid=(_NB3,),
        in_specs=[row_spec, row_spec, row_spec, full, vec, vec, vec, full,
                  vec,
                  pl.BlockSpec((1, 1, _BLK3), lambda i: (i, 0, 0)),
                  full, vec,
                  pl.BlockSpec((D, 10), lambda i: (0, 0)),
                  pl.BlockSpec((1, 10), lambda i: (0, 0))],
        out_specs=pl.BlockSpec((G, 10), lambda i: (0, 0)),
        out_shape=jax.ShapeDtypeStruct((G, 10), jnp.float32),
        scratch_shapes=[pltpu.VMEM((G, D), jnp.float32),
                        pltpu.VMEM((G, 1), jnp.float32)],
    )(h, p0, p1, w1, b1.reshape(1, D), g.reshape(1, D), be.reshape(1, D),
      w2, b2.reshape(1, D), batch.reshape(_NB3, 1, _BLK3),
      hw1, hb1.reshape(1, D), hw2, hb2.reshape(1, 10))


@jax.jit
def kernel(x, edge_index, batch, conv_W1, conv_b1, conv_gamma, conv_beta,
           conv_W2, conv_b2, head_W1, head_b1, head_W2, head_b2):
    src = edge_index[0].reshape(NC, NS, EPT)
    dst = edge_index[1].reshape(NC, NS, NCHUNK, CHUNK)
    zeros = jnp.zeros((RPT, D), dtype=jnp.float32)
    h0 = x
    for i in range(2):
        p = _sc_agg(h0, src, dst, zeros).reshape(NC, N, D)
        h0 = _tc_layer(h0, p[0], p[1], conv_W1[i], conv_b1[i],
                       conv_gamma[i], conv_beta[i], conv_W2[i], conv_b2[i])
    p = _sc_agg(h0, src, dst, zeros).reshape(NC, N, D)
    return _tc_layer3_pool_head(h0, p[0], p[1], conv_W1[2], conv_b1[2],
                                conv_gamma[2], conv_beta[2], conv_W2[2],
                                conv_b2[2], batch, head_W1, head_b1,
                                head_W2, head_b2)
